# baseline (speedup 1.0000x reference)
import functools

import jax
import jax.numpy as jnp
from jax import lax
from jax.experimental import pallas as pl
from jax.experimental.pallas import tpu as pltpu
from jax.experimental.pallas import tpu_sc as plsc

_info = plsc.get_sparse_core_info()
_NC, _NS = _info.num_cores, _info.num_subcores
_NW = _NC * _NS
_NPG = 4
_NTG = _NW // _NPG


def _sc_plane_gather(idx, wt):
    mesh = plsc.VectorSubcoreMesh(core_axis_name="c", subcore_axis_name="s")

    @functools.partial(
        pl.kernel,
        out_type=jax.ShapeDtypeStruct((256, 4, 8, 8, 128), jnp.float32),
        mesh=mesh,
        compiler_params=pltpu.CompilerParams(
            use_tc_tiling_on_sc=False, needs_layout_passes=False
        ),
        scratch_types=[
            pltpu.VMEM((8 * 8192,), jnp.float32),
            pltpu.VMEM((4, 8, 8, 128), jnp.int32),
            pltpu.VMEM((2, 8, 8, 128), jnp.float32),
            pltpu.SemaphoreType.DMA,
            pltpu.SemaphoreType.DMA,
        ],
    )
    def body(idx_hbm, wt_hbm, out_hbm, planes_v, idx_v, stage_v,
             wsem0, wsem1):
        wid = lax.axis_index("s") * _NC + lax.axis_index("c")
        tg = wid // _NPG
        pg = wid % _NPG
        pltpu.sync_copy(wt_hbm.at[pg], planes_v)
        pltpu.sync_copy(idx_hbm.at[pl.ds(4 * tg, 4)], idx_v)
        wsems = (wsem0, wsem1)
        zeros16 = [jnp.zeros((16,), jnp.float32) for _ in range(8)]

        def wait_write(slot):
            pltpu.make_async_copy(
                stage_v.at[slot], out_hbm.at[0].at[pg], wsems[slot]
            ).wait()

        def pair_body(i, carry):
            for slot in range(2):
                @pl.when(i > 0)
                def _():
                    wait_write(slot)
                bl = 2 * i + slot
                rt = lax.shift_right_logical(bl, 3)
                sb = lax.bitwise_and(bl, 7)

                def load_k(ct, l16):
                    vidx = idx_v[rt, ct, sb, pl.ds(16 * l16, 16)]
                    return lax.add(
                        lax.shift_left(lax.shift_right_logical(vidx, 7), 10),
                        lax.bitwise_and(vidx, 127),
                    )

                def ct_body(ct, prev):
                    ct_prev = lax.max(ct - 1, 0)
                    for l16 in range(8):
                        k = load_k(ct, l16)
                        cur = []
                        for s in range(8):
                            cur.append(plsc.load_gather(
                                planes_v.at[pl.ds(128 * s, 64640)], [k]
                            ))
                            if l16 == 0:
                                stage_v[slot, ct_prev, s, pl.ds(16 * 7, 16)] = prev[s]
                            else:
                                stage_v[slot, ct, s, pl.ds(16 * (l16 - 1), 16)] = prev[s]
                        prev = tuple(cur)
                    return prev

                last = lax.fori_loop(0, 8, ct_body, tuple(zeros16))
                for s in range(8):
                    stage_v[slot, 7, s, pl.ds(16 * 7, 16)] = last[s]
                b = 32 * tg + bl
                pltpu.async_copy(
                    stage_v.at[slot], out_hbm.at[b].at[pg], wsems[slot]
                )
            return carry

        lax.fori_loop(0, 16, pair_body, 0)
        wait_write(0)
        wait_write(1)

    return body(idx, wt)


def kernel(embed_id, weight):
    idx = embed_id.astype(jnp.int32).reshape(32, 8, 8, 128).transpose(0, 2, 1, 3)
    wt = (weight.reshape(64, 128, 4, 8).transpose(2, 0, 3, 1)
          .reshape(4, 8 * 8192))
    out5 = _sc_plane_gather(idx, wt)
    out = out5.transpose(0, 1, 3, 2, 4).reshape(256, 32, 1024).transpose(0, 2, 1)
    return out


# speedup vs baseline: 1.2832x; 1.2832x over previous
import functools

import jax
import jax.numpy as jnp
from jax import lax
from jax.experimental import pallas as pl
from jax.experimental.pallas import tpu as pltpu
from jax.experimental.pallas import tpu_sc as plsc

_info = plsc.get_sparse_core_info()
_NC, _NS = _info.num_cores, _info.num_subcores
_NW = _NC * _NS
_NPG = 4
_NTG = _NW // _NPG


def _sc_plane_gather(idx, wt):
    mesh = plsc.VectorSubcoreMesh(core_axis_name="c", subcore_axis_name="s")

    @functools.partial(
        pl.kernel,
        out_type=jax.ShapeDtypeStruct((256, 4, 8, 8, 128), jnp.float32),
        mesh=mesh,
        compiler_params=pltpu.CompilerParams(
            use_tc_tiling_on_sc=False, needs_layout_passes=False
        ),
        scratch_types=[
            pltpu.VMEM((8 * 8192,), jnp.float32),
            pltpu.VMEM((4, 8, 8, 128), jnp.int32),
            pltpu.VMEM((2, 8, 8, 128), jnp.float32),
            pltpu.SemaphoreType.DMA,
            pltpu.SemaphoreType.DMA,
        ],
    )
    def body(idx_hbm, wt_hbm, out_hbm, planes_v, idx_v, stage_v,
             wsem0, wsem1):
        wid = lax.axis_index("s") * _NC + lax.axis_index("c")
        tg = wid // _NPG
        pg = wid % _NPG
        pltpu.sync_copy(wt_hbm.at[pl.ds(8 * 8192 * pg, 8 * 8192)], planes_v)
        pltpu.sync_copy(idx_hbm.at[pl.ds(4 * tg, 4)], idx_v)
        wsems = (wsem0, wsem1)
        zeros16 = [jnp.zeros((16,), jnp.float32) for _ in range(8)]

        def wait_write(slot):
            pltpu.make_async_copy(
                stage_v.at[slot], out_hbm.at[0].at[pg], wsems[slot]
            ).wait()

        def pair_body(i, carry):
            for slot in range(2):
                @pl.when(i > 0)
                def _():
                    wait_write(slot)
                bl = 2 * i + slot
                rt = lax.shift_right_logical(bl, 3)
                sb = lax.bitwise_and(bl, 7)

                def ct_body(ct, prev):
                    ct_prev = lax.max(ct - 1, 0)
                    vidxs = []
                    for j in range(8):
                        vidxs.append(idx_v[rt, ct, sb, pl.ds(16 * j, 16)])
                        stage_v[slot, ct_prev, j, pl.ds(16 * 7, 16)] = prev[j]
                    grp = None
                    for l16 in range(8):
                        cur = []
                        for s in range(8):
                            cur.append(plsc.load_gather(
                                planes_v.at[pl.ds(8192 * s, 8192)], [vidxs[l16]]
                            ))
                            if l16 > 0:
                                stage_v[slot, ct, s, pl.ds(16 * (l16 - 1), 16)] = grp[s]
                        grp = cur
                    return tuple(grp)

                last = lax.fori_loop(0, 8, ct_body, tuple(zeros16))
                for s in range(8):
                    stage_v[slot, 7, s, pl.ds(16 * 7, 16)] = last[s]
                b = 32 * tg + bl
                pltpu.async_copy(
                    stage_v.at[slot], out_hbm.at[b].at[pg], wsems[slot]
                )
            return carry

        lax.fori_loop(0, 16, pair_body, 0)
        wait_write(0)
        wait_write(1)

    return body(idx, wt)


def kernel(embed_id, weight):
    idx = embed_id.astype(jnp.int32).reshape(32, 8, 8, 128).transpose(0, 2, 1, 3)
    wt = jnp.transpose(weight).reshape(-1)
    out5 = _sc_plane_gather(idx, wt)
    out = out5.transpose(0, 1, 3, 2, 4).reshape(256, 32, 1024).transpose(0, 2, 1)
    return out


# speedup vs baseline: 1.3196x; 1.0283x over previous
import functools

import jax
import jax.numpy as jnp
from jax import lax
from jax.experimental import pallas as pl
from jax.experimental.pallas import tpu as pltpu
from jax.experimental.pallas import tpu_sc as plsc

_info = plsc.get_sparse_core_info()
_NC, _NS = _info.num_cores, _info.num_subcores
_NW = _NC * _NS
_NPG = 4
_NTG = _NW // _NPG


def _sc_plane_gather(idx, wt):
    mesh = plsc.VectorSubcoreMesh(core_axis_name="c", subcore_axis_name="s")

    @functools.partial(
        pl.kernel,
        out_type=jax.ShapeDtypeStruct((256, 4, 8, 8, 128), jnp.float32),
        mesh=mesh,
        compiler_params=pltpu.CompilerParams(
            use_tc_tiling_on_sc=False, needs_layout_passes=False
        ),
        scratch_types=[
            pltpu.VMEM((8 * 8192,), jnp.float32),
            pltpu.VMEM((4, 8, 8, 128), jnp.int32),
            pltpu.VMEM((2, 8, 8, 128), jnp.float32),
            pltpu.SemaphoreType.DMA,
            pltpu.SemaphoreType.DMA,
        ],
    )
    def body(idx_hbm, wt_hbm, out_hbm, planes_v, idx_v, stage_v,
             wsem0, wsem1):
        wid = lax.axis_index("s") * _NC + lax.axis_index("c")
        tg = wid // _NPG
        pg = wid % _NPG
        cp_w = pltpu.async_copy(
            wt_hbm.at[pl.ds(8 * 8192 * pg, 8 * 8192)], planes_v, wsem0
        )
        cp_i = pltpu.async_copy(idx_hbm.at[pl.ds(4 * tg, 4)], idx_v, wsem1)
        cp_w.wait()
        cp_i.wait()
        wsems = (wsem0, wsem1)
        zeros16 = [jnp.zeros((16,), jnp.float32) for _ in range(8)]

        def wait_write(slot):
            pltpu.make_async_copy(
                stage_v.at[slot], out_hbm.at[0].at[pg], wsems[slot]
            ).wait()

        def pair_body(i, carry):
            for slot in range(2):
                @pl.when(i > 0)
                def _():
                    wait_write(slot)
                bl = 2 * i + slot
                rt = lax.shift_right_logical(bl, 3)
                sb = lax.bitwise_and(bl, 7)

                def ct_body(ct, prev):
                    ct_prev = lax.max(ct - 1, 0)
                    vidxs = []
                    for j in range(8):
                        vidxs.append(idx_v[rt, ct, sb, pl.ds(16 * j, 16)])
                        stage_v[slot, ct_prev, j, pl.ds(16 * 7, 16)] = prev[j]
                    grp = None
                    for l16 in range(8):
                        cur = []
                        for s in range(8):
                            cur.append(plsc.load_gather(
                                planes_v.at[pl.ds(8192 * s, 8192)], [vidxs[l16]]
                            ))
                            if l16 > 0:
                                stage_v[slot, ct, s, pl.ds(16 * (l16 - 1), 16)] = grp[s]
                        grp = cur
                    return tuple(grp)

                last = lax.fori_loop(0, 8, ct_body, tuple(zeros16))
                for s in range(8):
                    stage_v[slot, 7, s, pl.ds(16 * 7, 16)] = last[s]
                b = 32 * tg + bl
                pltpu.async_copy(
                    stage_v.at[slot], out_hbm.at[b].at[pg], wsems[slot]
                )
            return carry

        lax.fori_loop(0, 16, pair_body, 0)
        wait_write(0)
        wait_write(1)

    return body(idx, wt)


def kernel(embed_id, weight):
    idx = embed_id.astype(jnp.int32).reshape(32, 8, 8, 128).transpose(0, 2, 1, 3)
    wt = jnp.transpose(weight).reshape(-1)
    out5 = _sc_plane_gather(idx, wt)
    out = out5.transpose(0, 1, 3, 2, 4).reshape(256, 32, 1024).transpose(0, 2, 1)
    return out
